# Initial kernel scaffold; baseline (speedup 1.0000x reference)
import jax
import jax.numpy as jnp
from jax.experimental import pallas as pl


def kernel(input_pos, k_val, v_val, k_cache, v_cache):
    raise NotImplementedError("write your pallas kernel here")


# speedup vs baseline: 1.0567x; 1.0567x over previous
import jax
import jax.numpy as jnp
from jax.experimental import pallas as pl
from jax.experimental.pallas import tpu as pltpu

B, H, S, D, MAX_S = 8, 16, 16, 128, 4096


def _body(pos_ref, kv_ref, vv_ref, kc_ref, vc_ref, ko_ref, vo_ref):
    ko_ref[...] = kc_ref[...]
    vo_ref[...] = vc_ref[...]
    for s in range(S):
        p = pos_ref[s]
        ko_ref[0, pl.ds(p, 1), :] = kv_ref[0, pl.ds(s, 1), :]
        vo_ref[0, pl.ds(p, 1), :] = vv_ref[0, pl.ds(s, 1), :]


def kernel(input_pos, k_val, v_val, k_cache, v_cache):
    BH = B * H
    kv = k_val.reshape(BH, S, D)
    vv = v_val.reshape(BH, S, D)
    kc = k_cache.reshape(BH, MAX_S, D)
    vc = v_cache.reshape(BH, MAX_S, D)

    grid = (BH,)
    val_spec = pl.BlockSpec((1, S, D), lambda i: (i, 0, 0))
    cache_spec = pl.BlockSpec((1, MAX_S, D), lambda i: (i, 0, 0))
    pos_spec = pl.BlockSpec(memory_space=pltpu.SMEM)

    ko, vo = pl.pallas_call(
        _body,
        grid=grid,
        in_specs=[pos_spec, val_spec, val_spec, cache_spec, cache_spec],
        out_specs=[cache_spec, cache_spec],
        out_shape=[
            jax.ShapeDtypeStruct((BH, MAX_S, D), k_cache.dtype),
            jax.ShapeDtypeStruct((BH, MAX_S, D), v_cache.dtype),
        ],
    )(input_pos, kv, vv, kc, vc)

    return (ko.reshape(B, H, MAX_S, D), vo.reshape(B, H, MAX_S, D))
